# baseline (speedup 1.0000x reference)
import functools

import jax
import jax.numpy as jnp
from jax import lax
from jax.experimental import pallas as pl
from jax.experimental.pallas import tpu as pltpu
from jax.experimental.pallas import tpu_sc as plsc

ROWS = 128
COLS = 32768
LANES = 16
NUM_CORES = 2
NUM_SUBCORES = 16
NUM_WORKERS = NUM_CORES * NUM_SUBCORES
ROWS_PER_WORKER = ROWS // NUM_WORKERS
CHUNK = 4096
NUM_CHUNKS = COLS // CHUNK
VREGS_PER_CHUNK = CHUNK // LANES


def _sc_masked_cumsum(x_hbm, m_hbm, out_hbm,
                      xb0, xb1, mb0, mb1, sem_in0, sem_in1, sem_out):
    wid = lax.axis_index("s") * NUM_CORES + lax.axis_index("c")
    row0 = wid * ROWS_PER_WORKER
    xb = (xb0, xb1)
    mb = (mb0, mb1)
    sem_in = (sem_in0, sem_in1)

    def start_in(c, s):
        col = pl.ds(c * CHUNK, CHUNK)
        h = []
        for r in range(ROWS_PER_WORKER):
            h.append(pltpu.async_copy(x_hbm.at[row0 + r, col], xb[s].at[r],
                                      sem_in[s]))
            h.append(pltpu.async_copy(m_hbm.at[row0 + r, col], mb[s].at[r],
                                      sem_in[s]))
        return h

    def start_out(c, s):
        col = pl.ds(c * CHUNK, CHUNK)
        return [pltpu.async_copy(xb[s].at[r], out_hbm.at[row0 + r, col],
                                 sem_out)
                for r in range(ROWS_PER_WORKER)]

    carries = (jnp.zeros((LANES,), jnp.float32),) * ROWS_PER_WORKER
    in_h = {0: start_in(0, 0)}
    out_h = {}
    for c in range(NUM_CHUNKS):
        s = c & 1
        if c + 1 < NUM_CHUNKS:
            if c - 1 >= 0:
                for h in out_h.pop(c - 1):
                    h.wait()
            in_h[c + 1] = start_in(c + 1, 1 - s)
        for h in in_h.pop(c):
            h.wait()

        xbuf, mbuf = xb[s], mb[s]

        def body(j, carry, xbuf=xbuf, mbuf=mbuf):
            iota = lax.iota(jnp.int32, LANES)
            last = jnp.full((LANES,), LANES - 1, jnp.int32)
            base = j * LANES
            out = []
            for r in range(ROWS_PER_WORKER):
                sl = (r, pl.ds(base, LANES))
                v = xbuf[sl] * mbuf[sl]
                for k in (1, 2, 4, 8):
                    sh = jnp.take_along_axis(
                        v, jnp.maximum(iota - k, 0), axis=0,
                        mode="promise_in_bounds")
                    v = v + jnp.where(iota >= k, sh, 0.0)
                ov = v + carry[r]
                xbuf[sl] = ov
                out.append(jnp.take_along_axis(
                    ov, last, axis=0, mode="promise_in_bounds"))
            return tuple(out)

        carries = lax.fori_loop(0, VREGS_PER_CHUNK, body, carries, unroll=2)
        out_h[c] = start_out(c, s)
    for c in (NUM_CHUNKS - 2, NUM_CHUNKS - 1):
        for h in out_h.pop(c, ()):
            h.wait()


@jax.jit
def _masked_cumsum(x, mask_f32):
    mesh = plsc.VectorSubcoreMesh(core_axis_name="c", subcore_axis_name="s")
    kern = functools.partial(
        pl.kernel,
        out_type=jax.ShapeDtypeStruct((ROWS, COLS), jnp.float32),
        mesh=mesh,
        scratch_types=[
            pltpu.VMEM((ROWS_PER_WORKER, CHUNK), jnp.float32),
            pltpu.VMEM((ROWS_PER_WORKER, CHUNK), jnp.float32),
            pltpu.VMEM((ROWS_PER_WORKER, CHUNK), jnp.float32),
            pltpu.VMEM((ROWS_PER_WORKER, CHUNK), jnp.float32),
            pltpu.SemaphoreType.DMA,
            pltpu.SemaphoreType.DMA,
            pltpu.SemaphoreType.DMA,
        ],
        compiler_params=pltpu.CompilerParams(needs_layout_passes=False),
    )(_sc_masked_cumsum)
    return kern(x, mask_f32)


def kernel(x, mask):
    return _masked_cumsum(x, mask.astype(jnp.float32))


# speedup vs baseline: 1.0452x; 1.0452x over previous
import functools

import jax
import jax.numpy as jnp
from jax import lax
from jax.experimental import pallas as pl
from jax.experimental.pallas import tpu as pltpu
from jax.experimental.pallas import tpu_sc as plsc

ROWS = 128
COLS = 32768
LANES = 16
NUM_CORES = 2
NUM_SUBCORES = 16
NUM_WORKERS = NUM_CORES * NUM_SUBCORES

SC_ROWS = 32
TC_ROWS = ROWS - SC_ROWS
SC_RPW = SC_ROWS // NUM_WORKERS

CHUNK = 8192
NUM_CHUNKS = COLS // CHUNK
VREGS_PER_CHUNK = CHUNK // LANES

TC_BLOCK = 512


def _sc_masked_cumsum(x_hbm, m_hbm, out_hbm,
                      xb0, xb1, mb0, mb1, sem_in0, sem_in1, sem_out):
    wid = lax.axis_index("s") * NUM_CORES + lax.axis_index("c")
    row0 = wid * SC_RPW
    xb = (xb0, xb1)
    mb = (mb0, mb1)
    sem_in = (sem_in0, sem_in1)

    def start_in(c, s):
        col = pl.ds(c * CHUNK, CHUNK)
        h = []
        for r in range(SC_RPW):
            h.append(pltpu.async_copy(x_hbm.at[row0 + r, col], xb[s].at[r],
                                      sem_in[s]))
            h.append(pltpu.async_copy(m_hbm.at[row0 + r, col], mb[s].at[r],
                                      sem_in[s]))
        return h

    def start_out(c, s):
        col = pl.ds(c * CHUNK, CHUNK)
        return [pltpu.async_copy(xb[s].at[r], out_hbm.at[row0 + r, col],
                                 sem_out)
                for r in range(SC_RPW)]

    carries = (jnp.float32(0.0),) * SC_RPW
    in_h = {0: start_in(0, 0)}
    out_h = {}
    for c in range(NUM_CHUNKS):
        s = c & 1
        if c + 1 < NUM_CHUNKS:
            if c - 1 >= 0:
                for h in out_h.pop(c - 1):
                    h.wait()
            in_h[c + 1] = start_in(c + 1, 1 - s)
        for h in in_h.pop(c):
            h.wait()

        xbuf, mbuf = xb[s], mb[s]

        def body(j, carry, xbuf=xbuf, mbuf=mbuf):
            base = j * LANES
            out = []
            for r in range(SC_RPW):
                sl = (r, pl.ds(base, LANES))
                v = xbuf[sl] * mbuf[sl]
                sc = plsc.cumsum(v)
                xbuf[sl] = sc + carry[r]
                out.append(carry[r] + sc[LANES - 1])
            return tuple(out)

        carries = lax.fori_loop(0, VREGS_PER_CHUNK, body, carries, unroll=4)
        out_h[c] = start_out(c, s)
    for c in (NUM_CHUNKS - 2, NUM_CHUNKS - 1):
        for h in out_h.pop(c, ()):
            h.wait()


def _sc_call(x, m):
    mesh = plsc.VectorSubcoreMesh(core_axis_name="c", subcore_axis_name="s")
    kern = functools.partial(
        pl.kernel,
        out_type=jax.ShapeDtypeStruct((SC_ROWS, COLS), jnp.float32),
        mesh=mesh,
        scratch_types=[
            pltpu.VMEM((SC_RPW, CHUNK), jnp.float32),
            pltpu.VMEM((SC_RPW, CHUNK), jnp.float32),
            pltpu.VMEM((SC_RPW, CHUNK), jnp.float32),
            pltpu.VMEM((SC_RPW, CHUNK), jnp.float32),
            pltpu.SemaphoreType.DMA,
            pltpu.SemaphoreType.DMA,
            pltpu.SemaphoreType.DMA,
        ],
        compiler_params=pltpu.CompilerParams(needs_layout_passes=False),
    )(_sc_masked_cumsum)
    return kern(x, m)


def _tc_body(x_ref, m_ref, tri_ref, out_ref, carry_ref):
    i = pl.program_id(0)

    @pl.when(i == 0)
    def _():
        carry_ref[...] = jnp.zeros_like(carry_ref)

    masked = x_ref[...] * m_ref[...]
    s = jax.lax.dot_general(
        masked.astype(jnp.bfloat16), tri_ref[...],
        (((1,), (0,)), ((), ())),
        preferred_element_type=jnp.float32)
    carry = carry_ref[:, 0:1]
    out_ref[...] = s + carry
    carry_ref[:, 0:1] = carry + s[:, TC_BLOCK - 1:TC_BLOCK]


def _tc_call(x, m, tri):
    rows = x.shape[0]
    grid = COLS // TC_BLOCK
    return pl.pallas_call(
        _tc_body,
        grid=(grid,),
        in_specs=[
            pl.BlockSpec((rows, TC_BLOCK), lambda i: (0, i)),
            pl.BlockSpec((rows, TC_BLOCK), lambda i: (0, i)),
            pl.BlockSpec((TC_BLOCK, TC_BLOCK), lambda i: (0, 0)),
        ],
        out_specs=pl.BlockSpec((rows, TC_BLOCK), lambda i: (0, i)),
        out_shape=jax.ShapeDtypeStruct((rows, COLS), jnp.float32),
        scratch_shapes=[pltpu.VMEM((rows, 128), jnp.float32)],
    )(x, m, tri)


@jax.jit
def _masked_cumsum(x, mask_f32):
    tri = jnp.triu(jnp.ones((TC_BLOCK, TC_BLOCK), jnp.bfloat16))
    top = _sc_call(x[:SC_ROWS], mask_f32[:SC_ROWS])
    bot = _tc_call(x[SC_ROWS:], mask_f32[SC_ROWS:], tri)
    return jnp.concatenate([top, bot], axis=0)


def kernel(x, mask):
    return _masked_cumsum(x, mask.astype(jnp.float32))


# speedup vs baseline: 2.0102x; 1.9233x over previous
import jax
import jax.numpy as jnp
from jax.experimental import pallas as pl
from jax.experimental.pallas import tpu as pltpu

ROWS = 128
COLS = 32768
TC_BLOCK = 512


def _tc_body(x_ref, m_ref, tri_ref, out_ref, carry_ref):
    i = pl.program_id(0)

    @pl.when(i == 0)
    def _():
        carry_ref[...] = jnp.zeros_like(carry_ref)

    masked = jnp.where(m_ref[...], x_ref[...], 0.0)
    s = jax.lax.dot_general(
        masked.astype(jnp.bfloat16), tri_ref[...],
        (((1,), (0,)), ((), ())),
        preferred_element_type=jnp.float32)
    carry = carry_ref[:, 0:1]
    out_ref[...] = s + carry
    carry_ref[:, 0:1] = carry + s[:, TC_BLOCK - 1:TC_BLOCK]


@jax.jit
def _masked_cumsum(x, mask):
    tri = jnp.triu(jnp.ones((TC_BLOCK, TC_BLOCK), jnp.bfloat16))
    grid = COLS // TC_BLOCK
    return pl.pallas_call(
        _tc_body,
        grid=(grid,),
        in_specs=[
            pl.BlockSpec((ROWS, TC_BLOCK), lambda i: (0, i)),
            pl.BlockSpec((ROWS, TC_BLOCK), lambda i: (0, i)),
            pl.BlockSpec((TC_BLOCK, TC_BLOCK), lambda i: (0, 0)),
        ],
        out_specs=pl.BlockSpec((ROWS, TC_BLOCK), lambda i: (0, i)),
        out_shape=jax.ShapeDtypeStruct((ROWS, COLS), jnp.float32),
        scratch_shapes=[pltpu.VMEM((ROWS, 128), jnp.float32)],
    )(x, mask, tri)


def kernel(x, mask):
    return _masked_cumsum(x, mask)
